# baseline (speedup 1.0000x reference)
import functools
import jax
import jax.numpy as jnp
import numpy as np
from jax.experimental import pallas as pl

N = 10000
E = 320000
F = 128
GFS = 16
NC = 8
B = 64
HEADS = 5
DH = 2 * F

_SELU_L = 1.0507009873554805
_SELU_A = 1.6732632423543772
_BN_S = 1.0 / np.sqrt(1.0 + 1e-5)

_interp = False


def _selu(x):
    return _SELU_L * jnp.where(x > 0, x, _SELU_A * (jnp.exp(jnp.minimum(x, 0.0)) - 1.0))


def _mm_kernel(x_ref, w_ref, b_ref, pg_ref, pb_ref, qg_ref, qb_ref, o_ref,
               *, pre_bn, pre_selu, post_bn, post_selu, bias):
    x = x_ref[...]
    if pre_bn:
        x = pg_ref[...] * x + pb_ref[...]
    if pre_selu:
        x = _selu(x)
    y = jnp.dot(x, w_ref[...], preferred_element_type=jnp.float32)
    if bias:
        y = y + b_ref[...]
    if post_bn:
        y = qg_ref[...] * y + qb_ref[...]
    if post_selu:
        y = _selu(y)
    o_ref[...] = y


def _mm(x, w, b=None, pre=None, pre_selu=False, post=None, post_selu=False, bm=512):
    m, k = x.shape
    n = w.shape[1]
    assert m % bm == 0, (m, bm)
    one = jnp.zeros((1, 1), jnp.float32)
    b2 = one if b is None else b.reshape(1, n)
    pg, pb = (one, one) if pre is None else (pre[0].reshape(1, k), pre[1].reshape(1, k))
    qg, qb = (one, one) if post is None else (post[0].reshape(1, n), post[1].reshape(1, n))
    zspec = pl.BlockSpec((1, 1), lambda i: (0, 0))
    kspec = pl.BlockSpec((1, k), lambda i: (0, 0))
    nspec = pl.BlockSpec((1, n), lambda i: (0, 0))
    return pl.pallas_call(
        functools.partial(_mm_kernel, pre_bn=pre is not None, pre_selu=pre_selu,
                          post_bn=post is not None, post_selu=post_selu,
                          bias=b is not None),
        grid=(m // bm,),
        in_specs=[
            pl.BlockSpec((bm, k), lambda i: (i, 0)),
            pl.BlockSpec((k, n), lambda i: (0, 0)),
            nspec if b is not None else zspec,
            kspec if pre is not None else zspec,
            kspec if pre is not None else zspec,
            nspec if post is not None else zspec,
            nspec if post is not None else zspec,
        ],
        out_specs=pl.BlockSpec((bm, n), lambda i: (i, 0)),
        out_shape=jax.ShapeDtypeStruct((m, n), jnp.float32),
        interpret=_interp,
    )(x, w, b2, pg, pb, qg, qb)


def _ew_kernel(x_ref, g_ref, b_ref, o_ref):
    o_ref[...] = _selu(g_ref[...] * x_ref[...] + b_ref[...])


def _ew_selu_bn(x, g, b, bm=400):
    m, k = x.shape
    return pl.pallas_call(
        _ew_kernel,
        grid=(m // bm,),
        in_specs=[pl.BlockSpec((bm, k), lambda i: (i, 0)),
                  pl.BlockSpec((1, k), lambda i: (0, 0)),
                  pl.BlockSpec((1, k), lambda i: (0, 0))],
        out_specs=pl.BlockSpec((bm, k), lambda i: (i, 0)),
        out_shape=jax.ShapeDtypeStruct((m, k), jnp.float32),
        interpret=_interp,
    )(x, g.reshape(1, k), b.reshape(1, k))


def _bn_fold(g, bt, extra_bias=None):
    gs = g * _BN_S
    sh = bt if extra_bias is None else gs * extra_bias + bt
    return gs, sh


def _gat_alpha(a_s, a_d, srcs, dsts, gat_w):
    z = a_s[srcs] + a_d[dsts]
    z = jnp.where(z > 0, z, 0.2 * z)
    amax = jax.ops.segment_max(z, dsts, num_segments=N)
    w = jnp.exp(z - amax[dsts]) * gat_w[:, None]
    denom = jax.ops.segment_sum(w, dsts, num_segments=N)
    return w / (denom[dsts] + 1e-16)


def kernel(x, u, edge_attr, params, edge_index, batch, y):
    del edge_attr, y
    p = params
    src = edge_index[0]
    dst = edge_index[1]

    loop = jnp.arange(N, dtype=dst.dtype)
    dst_all = jnp.concatenate([dst, loop])
    src_all = jnp.concatenate([src, loop])
    ecw_all = jnp.concatenate([jnp.ones((E,), jnp.float32), jnp.zeros((N,), jnp.float32)])
    order = jnp.argsort(dst_all)
    EP = 330240
    npad = EP - (E + N)
    dsts = jnp.concatenate([dst_all[order], jnp.full((npad,), N - 1, dst.dtype)])
    srcs = jnp.concatenate([src_all[order], jnp.zeros((npad,), dst.dtype)])
    ecw = jnp.concatenate([ecw_all[order], jnp.zeros((npad,), jnp.float32)])
    gat_w = jnp.concatenate([jnp.ones((E + N,), jnp.float32), jnp.zeros((npad,), jnp.float32)])

    h1 = _mm(x, p['gat1_W'], bm=400)
    A1 = jnp.zeros((HEADS * DH, 128), jnp.float32)
    for hh in range(HEADS):
        A1 = A1.at[hh * DH:(hh + 1) * DH, hh].set(p['gat1_as'][hh])
        A1 = A1.at[hh * DH:(hh + 1) * DH, HEADS + hh].set(p['gat1_ad'][hh])
    a1 = _mm(h1, A1, bm=400)
    alpha1 = _gat_alpha(a1[:, :HEADS], a1[:, HEADS:2 * HEADS], srcs, dsts, gat_w)
    msg1 = h1.reshape(N, HEADS, DH)[srcs] * alpha1[:, :, None]
    out1 = jax.ops.segment_sum(msg1, dsts, num_segments=N).reshape(N, HEADS * DH)

    pre1 = _bn_fold(p['bn1_g'], p['bn1_b'], p['gat1_b'])
    h3 = _mm(out1, p['gat2_W'], pre=pre1, pre_selu=True, bm=400)
    A2 = jnp.zeros((DH, 128), jnp.float32)
    A2 = A2.at[:, 0].set(p['gat2_as'][0])
    A2 = A2.at[:, 1].set(p['gat2_ad'][0])
    a2 = _mm(h3, A2, bm=400)
    alpha2 = _gat_alpha(a2[:, 0:1], a2[:, 1:2], srcs, dsts, gat_w)
    out2 = jax.ops.segment_sum(h3[srcs] * alpha2, dsts, num_segments=N)

    cnt = jax.ops.segment_sum(ecw, dsts, num_segments=N)
    inv_cnt = 1.0 / jnp.maximum(cnt, 1.0)

    g2, s2 = _bn_fold(p['bn2_g'], p['bn2_b'], p['gat2_b'])
    h4 = _ew_selu_bn(out2, g2, s2)

    def edge_conv(h, layers):
        d = h.shape[1]
        (W1, b1, g1, t1), (W2, b2, g2_, t2), (W3, b3, g3, t3) = layers
        P = _mm(h, W1[:d], bm=400)
        Q = _mm(h, W1[d:], bm=400)
        g1pre = P[dsts] + Q[srcs]
        e2 = _mm(g1pre, W2, b=b2, pre=_bn_fold(g1, t1, b1), pre_selu=True,
                 post=_bn_fold(g2_, t2), post_selu=True)
        m = _mm(e2, W3, b=b3, post=_bn_fold(g3, t3), post_selu=True)
        s = jax.ops.segment_sum(m * ecw[:, None], dsts, num_segments=N)
        return s * inv_cnt[:, None]

    agg1 = edge_conv(h4, p['mlp1'])
    h5 = jnp.concatenate([agg1, h4], axis=1)
    agg2 = edge_conv(h5, p['mlp2'])
    h6 = jnp.concatenate([agg2, h5], axis=1)

    s = jax.ops.segment_sum(h6, batch, num_segments=B)
    bc = jax.ops.segment_sum(jnp.ones((N,), jnp.float32), batch, num_segments=B)
    g = s / jnp.maximum(bc, 1.0)[:, None]
    g = jnp.concatenate([g, u], axis=1)
    KH = 896
    gpad = jnp.concatenate([g, jnp.zeros((B, KH - g.shape[1]), jnp.float32)], axis=1)
    n1g = jnp.concatenate([p['n1_g'] * _BN_S, jnp.zeros((KH - g.shape[1],), jnp.float32)])
    n1b = jnp.concatenate([p['n1_b'], jnp.zeros((KH - g.shape[1],), jnp.float32)])
    fc1Wp = jnp.concatenate([p['fc1_W'], jnp.zeros((KH - g.shape[1], 256), jnp.float32)], axis=0)
    g = _mm(gpad, fc1Wp, b=p['fc1_b'], pre=(n1g, n1b), post_selu=True, bm=64)
    fc2Wp = jnp.concatenate([p['fc2_W'], jnp.zeros((256, 128 - NC), jnp.float32)], axis=1)
    fc2bp = jnp.concatenate([p['fc2_b'], jnp.zeros((128 - NC,), jnp.float32)])
    out = _mm(g, fc2Wp, b=fc2bp, pre=(p['n2_g'] * _BN_S, p['n2_b']), bm=64)
    return out[:, :NC]


# speedup vs baseline: 3.3079x; 3.3079x over previous
import functools
import jax
import jax.numpy as jnp
import numpy as np
from jax import lax
from jax.experimental import pallas as pl
from jax.experimental.pallas import tpu as pltpu
from jax.experimental.pallas import tpu_sc as plsc

N = 10000
E = 320000
F = 128
GFS = 16
NC = 8
B = 64
HEADS = 5
DH = 2 * F

_SELU_L = 1.0507009873554805
_SELU_A = 1.6732632423543772
_BN_S = 1.0 / np.sqrt(1.0 + 1e-5)

_interp = False


def _selu(x):
    return _SELU_L * jnp.where(x > 0, x, _SELU_A * (jnp.exp(jnp.minimum(x, 0.0)) - 1.0))


def _mm_kernel(x_ref, w_ref, b_ref, pg_ref, pb_ref, qg_ref, qb_ref, o_ref,
               *, pre_bn, pre_selu, post_bn, post_selu, bias):
    x = x_ref[...]
    if pre_bn:
        x = pg_ref[...] * x + pb_ref[...]
    if pre_selu:
        x = _selu(x)
    y = jnp.dot(x, w_ref[...], preferred_element_type=jnp.float32)
    if bias:
        y = y + b_ref[...]
    if post_bn:
        y = qg_ref[...] * y + qb_ref[...]
    if post_selu:
        y = _selu(y)
    o_ref[...] = y


def _mm(x, w, b=None, pre=None, pre_selu=False, post=None, post_selu=False, bm=512):
    m, k = x.shape
    n = w.shape[1]
    assert m % bm == 0, (m, bm)
    one = jnp.zeros((1, 1), jnp.float32)
    b2 = one if b is None else b.reshape(1, n)
    pg, pb = (one, one) if pre is None else (pre[0].reshape(1, k), pre[1].reshape(1, k))
    qg, qb = (one, one) if post is None else (post[0].reshape(1, n), post[1].reshape(1, n))
    zspec = pl.BlockSpec((1, 1), lambda i: (0, 0))
    kspec = pl.BlockSpec((1, k), lambda i: (0, 0))
    nspec = pl.BlockSpec((1, n), lambda i: (0, 0))
    return pl.pallas_call(
        functools.partial(_mm_kernel, pre_bn=pre is not None, pre_selu=pre_selu,
                          post_bn=post is not None, post_selu=post_selu,
                          bias=b is not None),
        grid=(m // bm,),
        in_specs=[
            pl.BlockSpec((bm, k), lambda i: (i, 0)),
            pl.BlockSpec((k, n), lambda i: (0, 0)),
            nspec if b is not None else zspec,
            kspec if pre is not None else zspec,
            kspec if pre is not None else zspec,
            nspec if post is not None else zspec,
            nspec if post is not None else zspec,
        ],
        out_specs=pl.BlockSpec((bm, n), lambda i: (i, 0)),
        out_shape=jax.ShapeDtypeStruct((m, n), jnp.float32),
        interpret=_interp,
    )(x, w, b2, pg, pb, qg, qb)


def _ew_kernel(x_ref, g_ref, b_ref, o_ref):
    o_ref[...] = _selu(g_ref[...] * x_ref[...] + b_ref[...])


def _ew_selu_bn(x, g, b, bm=400):
    m, k = x.shape
    return pl.pallas_call(
        _ew_kernel,
        grid=(m // bm,),
        in_specs=[pl.BlockSpec((bm, k), lambda i: (i, 0)),
                  pl.BlockSpec((1, k), lambda i: (0, 0)),
                  pl.BlockSpec((1, k), lambda i: (0, 0))],
        out_specs=pl.BlockSpec((bm, k), lambda i: (i, 0)),
        out_shape=jax.ShapeDtypeStruct((m, k), jnp.float32),
        interpret=_interp,
    )(x, g.reshape(1, k), b.reshape(1, k))


_NW = 32
_EPAD = 331776
_PERW = _EPAD // _NW
_C = 128
_SC_MESH = dict(core_axis_name="c", subcore_axis_name="s")


def _wid():
    return lax.axis_index("s") * 2 + lax.axis_index("c")


def _sc_gather2add(P, Q, di, si):
    @functools.partial(
        pl.kernel,
        out_type=jax.ShapeDtypeStruct((_EPAD, 256), jnp.float32),
        mesh=plsc.VectorSubcoreMesh(**_SC_MESH),
        scratch_types=[
            pltpu.VMEM((_C,), jnp.int32),
            pltpu.VMEM((_C,), jnp.int32),
            pltpu.VMEM((_C, 256), jnp.float32),
            pltpu.VMEM((_C, 256), jnp.float32),
            pltpu.SemaphoreType.DMA,
            pltpu.SemaphoreType.DMA,
        ],
        compiler_params=pltpu.CompilerParams(use_tc_tiling_on_sc=False,
                                             needs_layout_passes=False),
    )
    def k(p_hbm, q_hbm, di_hbm, si_hbm, out_hbm, di_v, si_v, ra, rb, sem1, sem2):
        base = _wid() * _PERW

        def chunk(g, carry):
            off = base + g * _C
            pltpu.sync_copy(di_hbm.at[pl.ds(off, _C)], di_v)
            pltpu.sync_copy(si_hbm.at[pl.ds(off, _C)], si_v)
            cp1 = pltpu.async_copy(p_hbm.at[di_v], ra, sem1)
            cp2 = pltpu.async_copy(q_hbm.at[si_v], rb, sem2)
            cp1.wait()
            cp2.wait()

            def row(i, c):
                ivec = jnp.full((16,), i, jnp.int32)
                iota = lax.iota(jnp.int32, 16)
                for v in range(16):
                    col = iota + v * 16
                    a = plsc.load_gather(ra, [ivec, col])
                    b = plsc.load_gather(rb, [ivec, col])
                    plsc.store_scatter(ra, [ivec, col], a + b)
                return c

            lax.fori_loop(0, _C, row, 0)
            pltpu.sync_copy(ra, out_hbm.at[pl.ds(off, _C)])
            return carry

        lax.fori_loop(0, _PERW // _C, chunk, 0)

    return k(P, Q, di, si)


_SR = 32
_LBR = _C + 8


_GDN = lax.GatherDimensionNumbers(offset_dims=(), collapsed_slice_dims=(0,),
                                  start_index_map=(0,))


def _dg(x, idx):
    return lax.gather(x, idx[:, None], _GDN, slice_sizes=(1,),
                      mode=lax.GatherScatterMode.PROMISE_IN_BOUNDS)


def _sc_segsum(dsts, flags, mode, rows=None, table=None, ad=None, srcs=None):
    gat = mode == 'gat'
    D = 272 if gat else 256
    f32, i32 = jnp.float32, jnp.int32

    scratch = [
        pltpu.VMEM((_C,), i32),
        pltpu.VMEM((_C,), f32),
        pltpu.VMEM((_C, D), f32),
        pltpu.VMEM((_LBR * 256,), f32),
        pltpu.VMEM((_LBR * 16,), f32),
        pltpu.VMEM((256,), f32), pltpu.VMEM((16,), f32), pltpu.VMEM((16,), i32),
        pltpu.VMEM((256,), f32), pltpu.VMEM((16,), f32), pltpu.VMEM((16,), i32),
        pltpu.SMEM((8,), i32),
        pltpu.SemaphoreType.DMA,
        pltpu.SemaphoreType.DMA,
    ]
    if gat:
        scratch += [pltpu.VMEM((_C,), i32), pltpu.VMEM((_C, 16), f32)]

    out_type = [
        jax.ShapeDtypeStruct((N + 8, 256), f32),
        jax.ShapeDtypeStruct((2 * _NW, 256), f32),
        jax.ShapeDtypeStruct((2 * _NW, 16), f32),
        jax.ShapeDtypeStruct((2 * _NW, 16), i32),
    ]

    def body(*refs):
        if gat:
            (tab_h, ad_h, si_h, di_h, fl_h, out_h, sr_h, sdv_h, sid_h,
             ids_v, fl_v, rv, lb, db, sa, sad, said, sb, sbd, sbid,
             scal, sem1, sem2, si_v, adv) = refs
        else:
            (rows_h, di_h, fl_h, out_h, sr_h, sdv_h, sid_h,
             ids_v, fl_v, rv, lb, db, sa, sad, said, sb, sbd, sbid,
             scal, sem1, sem2) = refs
        w = _wid()
        base = w * _PERW
        zf = jnp.zeros((16,), f32)
        iota = lax.iota(i32, 16)
        c0 = jnp.zeros((16,), i32)
        c15 = jnp.full((16,), 15, i32)
        lane0 = (iota == 0).astype(f32)
        cj = [jnp.full((16,), j, i32) for j in range(16)]

        def zrow(r, c):
            for v in range(16):
                lb[pl.ds(r * 256 + v * 16, 16)] = zf
            db[pl.ds(r * 16, 16)] = zf
            return c
        lax.fori_loop(0, _LBR, zrow, 0)

        pltpu.sync_copy(di_h.at[pl.ds(base, 16)], ids_v.at[pl.ds(0, 16)])
        first = jnp.max(_dg(ids_v[pl.ds(0, 16)], c0))
        scal[2] = 0
        scal[3] = first

        def chunk(g, carry):
            off = base + g * _C
            pltpu.sync_copy(di_h.at[pl.ds(off, _C)], ids_v)
            pltpu.sync_copy(fl_h.at[pl.ds(off, _C)], fl_v)
            if gat:
                pltpu.sync_copy(si_h.at[pl.ds(off, _C)], si_v)
                cp1 = pltpu.async_copy(tab_h.at[si_v], rv, sem1)
                cp2 = pltpu.async_copy(ad_h.at[ids_v], adv, sem2)
                cp1.wait()
                cp2.wait()
            else:
                pltpu.sync_copy(rows_h.at[pl.ds(off, _C)], rv)
            cur_base = scal[3]
            cb = jnp.full((16,), cur_base, i32)

            def group(t, c):
                dvec = ids_v[pl.ds(t * 16, 16)] - cb
                fvec = fl_v[pl.ds(t * 16, 16)]
                for j in range(16):
                    evec = jnp.full((16,), t * 16 + j, i32)
                    ld = _dg(dvec, cj[j])
                    fs = _dg(fvec, cj[j])
                    if gat:
                        a_s = plsc.load_gather(rv, [evec, jnp.full((16,), 256, i32)])
                        a_d = plsc.load_gather(adv, [evec, c0])
                        z = a_s + a_d
                        z = jnp.where(z > 0, z, 0.2 * z)
                        wv = jnp.exp(z) * fs
                    else:
                        wv = fs
                    b16 = ld * 256
                    for v in range(16):
                        col = iota + v * 16
                        rvv = plsc.load_gather(rv, [evec, col])
                        plsc.addupdate_scatter(lb, [b16 + col], wv * rvv)
                    plsc.addupdate_scatter(db, [ld * 16 + iota], wv * lane0)
                return c
            lax.fori_loop(0, _C // 16, group, 0)

            last = jnp.max(_dg(ids_v[pl.ds(_C - 16, 16)], c15) - cb)

            def emit(r, c):
                dvv = db[pl.ds(r * 16, 16)]
                div = _dg(dvv, c0)
                if gat:
                    div = div + 1e-16
                else:
                    div = jnp.maximum(div, 1.0)
                emitted = scal[2]

                @pl.when(emitted == 0)
                def _():
                    for v in range(16):
                        sa[pl.ds(v * 16, 16)] = lb[pl.ds(r * 256 + v * 16, 16)]
                    sad[pl.ds(0, 16)] = dvv
                    said[pl.ds(0, 16)] = cb + r
                    scal[2] = 1

                @pl.when(emitted > 0)
                def _():
                    for v in range(16):
                        lb[pl.ds(r * 256 + v * 16, 16)] = lb[pl.ds(r * 256 + v * 16, 16)] / div
                    pltpu.sync_copy(lb.at[pl.ds(r * 256, 256)], out_h.at[cur_base + r])

                for v in range(16):
                    lb[pl.ds(r * 256 + v * 16, 16)] = zf
                db[pl.ds(r * 16, 16)] = zf
                return c
            lax.fori_loop(0, last, emit, 0)

            @pl.when(last > 0)
            def _():
                for v in range(16):
                    lb[pl.ds(v * 16, 16)] = lb[pl.ds(last * 256 + v * 16, 16)]
                    lb[pl.ds(last * 256 + v * 16, 16)] = zf
                db[pl.ds(0, 16)] = db[pl.ds(last * 16, 16)]
                db[pl.ds(last * 16, 16)] = zf

            scal[3] = cur_base + last
            return carry

        lax.fori_loop(0, _PERW // _C, chunk, 0)

        cbf = jnp.full((16,), scal[3], i32)

        @pl.when(scal[2] == 0)
        def _():
            for v in range(16):
                sa[pl.ds(v * 16, 16)] = lb[pl.ds(v * 16, 16)]
                sb[pl.ds(v * 16, 16)] = zf
            sad[pl.ds(0, 16)] = db[pl.ds(0, 16)]
            said[pl.ds(0, 16)] = cbf
            sbd[pl.ds(0, 16)] = zf
            sbid[pl.ds(0, 16)] = cbf

        @pl.when(scal[2] > 0)
        def _():
            for v in range(16):
                sb[pl.ds(v * 16, 16)] = lb[pl.ds(v * 16, 16)]
            sbd[pl.ds(0, 16)] = db[pl.ds(0, 16)]
            sbid[pl.ds(0, 16)] = cbf

        pltpu.sync_copy(sa, sr_h.at[2 * w])
        pltpu.sync_copy(sb, sr_h.at[2 * w + 1])
        pltpu.sync_copy(sad, sdv_h.at[2 * w])
        pltpu.sync_copy(sbd, sdv_h.at[2 * w + 1])
        pltpu.sync_copy(said, sid_h.at[2 * w])
        pltpu.sync_copy(sbid, sid_h.at[2 * w + 1])

    k = pl.kernel(body, out_type=out_type,
                  mesh=plsc.VectorSubcoreMesh(**_SC_MESH),
                  scratch_types=scratch,
                  compiler_params=pltpu.CompilerParams(use_tc_tiling_on_sc=False,
                                                       needs_layout_passes=False))
    if gat:
        return k(table, ad, srcs, dsts, flags)
    return k(rows, dsts, flags)


def _merge_kernel(ids_ref, rows_ref, divs_ref, main_ref, o_ref, *, ec, bm):
    i = pl.program_id(0)
    ids = ids_ref[...][:, 0:1]
    rows = rows_ref[...]
    divs = divs_ref[...]
    rel = ids - i * bm
    pos = jax.lax.broadcasted_iota(jnp.int32, (1, bm), 1)
    onehot = (rel == pos).astype(jnp.float32)
    sums = jax.lax.dot_general(onehot, rows, (((0,), (0,)), ((), ())),
                               preferred_element_type=jnp.float32)
    dv = jax.lax.dot_general(onehot, divs, (((0,), (0,)), ((), ())),
                             preferred_element_type=jnp.float32)
    pres = jnp.sum(onehot, axis=0)[:, None]
    d = dv[:, 0:1]
    d = jnp.maximum(d, 1.0) if ec else d + 1e-16
    o_ref[...] = jnp.where(pres > 0, sums / d, main_ref[...])


def _merge(seg_out, ec):
    main, srows, sdiv, sid = seg_out
    bm = 400
    return pl.pallas_call(
        functools.partial(_merge_kernel, ec=ec, bm=bm),
        grid=(N // bm,),
        in_specs=[pl.BlockSpec((2 * _NW, 16), lambda i: (0, 0)),
                  pl.BlockSpec((2 * _NW, 256), lambda i: (0, 0)),
                  pl.BlockSpec((2 * _NW, 16), lambda i: (0, 0)),
                  pl.BlockSpec((bm, 256), lambda i: (i, 0))],
        out_specs=pl.BlockSpec((bm, 256), lambda i: (i, 0)),
        out_shape=jax.ShapeDtypeStruct((N, 256), jnp.float32),
        interpret=_interp,
    )(sid, srows, sdiv, main)


def _pool_kernel(h_ref, b_ref, o_ref, *, bm):
    i = pl.program_id(0)
    blk = h_ref[...]
    bids = b_ref[...]
    pos = jax.lax.broadcasted_iota(jnp.int32, (1, B), 1)
    onehot = (bids == pos).astype(jnp.float32)
    s = jax.lax.dot_general(onehot, blk, (((0,), (0,)), ((), ())),
                            preferred_element_type=jnp.float32)
    c = jnp.sum(onehot, axis=0)[:, None]
    sf = jnp.concatenate([s, c, jnp.zeros((B, 127), jnp.float32)], axis=1)

    @pl.when(i == 0)
    def _():
        o_ref[...] = sf

    @pl.when(i > 0)
    def _():
        o_ref[...] = o_ref[...] + sf


def _pool(h6, batch):
    bm = 400
    return pl.pallas_call(
        functools.partial(_pool_kernel, bm=bm),
        grid=(N // bm,),
        in_specs=[pl.BlockSpec((bm, 768), lambda i: (i, 0)),
                  pl.BlockSpec((bm, 1), lambda i: (i, 0))],
        out_specs=pl.BlockSpec((B, 896), lambda i: (0, 0)),
        out_shape=jax.ShapeDtypeStruct((B, 896), jnp.float32),
        interpret=_interp,
    )(h6, batch.reshape(N, 1).astype(jnp.int32))


def _bn_fold(g, bt, extra_bias=None):
    gs = g * _BN_S
    sh = bt if extra_bias is None else gs * extra_bias + bt
    return gs, sh


def _gat_aug_weights(W, att_s, att_d, h):
    Wh = W[:, h * DH:(h + 1) * DH]
    k = W.shape[0]
    aug = jnp.concatenate([Wh, (Wh @ att_s[h])[:, None],
                           jnp.zeros((k, 15), jnp.float32)], axis=1)
    adw = jnp.concatenate([(Wh @ att_d[h])[:, None],
                           jnp.zeros((k, 15), jnp.float32)], axis=1)
    return aug, adw


def kernel(x, u, edge_attr, params, edge_index, batch, y):
    del edge_attr, y
    p = params
    src = edge_index[0]
    dst = edge_index[1]

    loop = jnp.arange(N, dtype=dst.dtype)
    dst_all = jnp.concatenate([dst, loop])
    src_all = jnp.concatenate([src, loop])
    ecw_all = jnp.concatenate([jnp.ones((E,), jnp.float32), jnp.zeros((N,), jnp.float32)])
    order = jnp.argsort(dst_all)
    EP = _EPAD
    npad = EP - (E + N)
    dsts = jnp.concatenate([dst_all[order], jnp.full((npad,), N - 1, dst.dtype)])
    srcs = jnp.concatenate([src_all[order], jnp.zeros((npad,), dst.dtype)])
    ecw = jnp.concatenate([ecw_all[order], jnp.zeros((npad,), jnp.float32)])
    gat_w = jnp.concatenate([jnp.ones((E + N,), jnp.float32), jnp.zeros((npad,), jnp.float32)])

    heads_out = []
    for hh in range(HEADS):
        aug, adw = _gat_aug_weights(p['gat1_W'], p['gat1_as'], p['gat1_ad'], hh)
        tab = _mm(x, aug, bm=400)
        adh = _mm(x, adw, bm=400)
        seg = _sc_segsum(dsts, gat_w, 'gat', table=tab, ad=adh, srcs=srcs)
        heads_out.append(_merge(seg, ec=False))
    out1 = jnp.concatenate(heads_out, axis=1)

    pre1 = _bn_fold(p['bn1_g'], p['bn1_b'], p['gat1_b'])
    aug2, adw2 = _gat_aug_weights(p['gat2_W'], p['gat2_as'], p['gat2_ad'], 0)
    tab2 = _mm(out1, aug2, pre=pre1, pre_selu=True, bm=400)
    ad2 = _mm(out1, adw2, pre=pre1, pre_selu=True, bm=400)
    seg2 = _sc_segsum(dsts, gat_w, 'gat', table=tab2, ad=ad2, srcs=srcs)
    out2 = _merge(seg2, ec=False)

    g2, s2 = _bn_fold(p['bn2_g'], p['bn2_b'], p['gat2_b'])
    h4 = _ew_selu_bn(out2, g2, s2)

    def edge_conv(h, layers):
        d = h.shape[1]
        (W1, b1, g1, t1), (W2, b2, g2_, t2), (W3, b3, g3, t3) = layers
        P = _mm(h, W1[:d], bm=400)
        Q = _mm(h, W1[d:], bm=400)
        g1pre = _sc_gather2add(P, Q, dsts, srcs)
        e2 = _mm(g1pre, W2, b=b2, pre=_bn_fold(g1, t1, b1), pre_selu=True,
                 post=_bn_fold(g2_, t2), post_selu=True)
        m = _mm(e2, W3, b=b3, post=_bn_fold(g3, t3), post_selu=True)
        return _merge(_sc_segsum(dsts, ecw, 'ec', rows=m), ec=True)

    agg1 = edge_conv(h4, p['mlp1'])
    h5 = jnp.concatenate([agg1, h4], axis=1)
    agg2 = edge_conv(h5, p['mlp2'])
    h6 = jnp.concatenate([agg2, h5], axis=1)

    sums = _pool(h6, batch)
    g = sums[:, :768] / jnp.maximum(sums[:, 768:769], 1.0)
    g = jnp.concatenate([g, u], axis=1)
    KH = 896
    gpad = jnp.concatenate([g, jnp.zeros((B, KH - g.shape[1]), jnp.float32)], axis=1)
    n1g = jnp.concatenate([p['n1_g'] * _BN_S, jnp.zeros((KH - g.shape[1],), jnp.float32)])
    n1b = jnp.concatenate([p['n1_b'], jnp.zeros((KH - g.shape[1],), jnp.float32)])
    fc1Wp = jnp.concatenate([p['fc1_W'], jnp.zeros((KH - g.shape[1], 256), jnp.float32)], axis=0)
    g = _mm(gpad, fc1Wp, b=p['fc1_b'], pre=(n1g, n1b), post_selu=True, bm=64)
    fc2Wp = jnp.concatenate([p['fc2_W'], jnp.zeros((256, 128 - NC), jnp.float32)], axis=1)
    fc2bp = jnp.concatenate([p['fc2_b'], jnp.zeros((128 - NC,), jnp.float32)])
    out = _mm(g, fc2Wp, b=fc2bp, pre=(p['n2_g'] * _BN_S, p['n2_b']), bm=64)
    return out[:, :NC]
